# Initial kernel scaffold; baseline (speedup 1.0000x reference)
import jax
import jax.numpy as jnp
from jax.experimental import pallas as pl


def kernel(x, gaussian_colors, gaussian_positions, gaussian_scales, gaussian_rotations, wave_coefficients):
    raise NotImplementedError("write your pallas kernel here")


# speedup vs baseline: 13.7295x; 13.7295x over previous
import jax
import jax.numpy as jnp
from jax.experimental import pallas as pl
from jax.experimental.pallas import tpu as pltpu

_K = 16
_F = 1024
_MAXF = 1024.0


def _topk_body(w_ref, vals_ref, freqs_ref):
    v = w_ref[...]
    a = jnp.abs(v)
    iota = jax.lax.broadcasted_iota(jnp.int32, a.shape, 1)
    vals = []
    idxs = []
    for _ in range(_K):
        m = jnp.max(a, axis=1, keepdims=True)
        idx = jnp.min(jnp.where(a >= m, iota, _F), axis=1, keepdims=True)
        sel = iota == idx
        vals.append(jnp.sum(jnp.where(sel, v, 0.0), axis=1, keepdims=True))
        idxs.append(idx)
        a = jnp.where(sel, -1.0, a)
    vals_ref[...] = jnp.concatenate(vals, axis=1)
    freqs_ref[...] = (_MAXF / _F) * jnp.concatenate(idxs, axis=1).astype(jnp.float32)


def _render_body(xt_ref, colors_ref, pos_ref, scales_ref, rot_ref,
                 vals_ref, freqs_ref, out_ref):
    xx = xt_ref[0:1, :]
    xy = xt_ref[1:2, :]
    px = pos_ref[:, 0:1]
    py = pos_ref[:, 1:2]
    relx = xx - px
    rely = xy - py
    rot = rot_ref[:, 0:1]
    c = jnp.cos(rot)
    s = jnp.sin(rot)
    tx = c * relx + s * rely
    ty = -s * relx + c * rely
    sx = scales_ref[:, 0:1]
    sy = scales_ref[:, 1:2]
    env = jnp.exp(-0.5 * ((tx * sx) ** 2 + (ty * sy) ** 2))
    vals = vals_ref[...]
    freqs = freqs_ref[...]
    twopi = 2.0 * jnp.pi
    wave_x = jnp.zeros_like(tx)
    wave_y = jnp.zeros_like(ty)
    for k in range(_K):
        wave_x = wave_x + vals[:, k:k + 1] * jnp.cos((twopi * freqs[:, k:k + 1]) * tx)
        ky = _K + k
        wave_y = wave_y + vals[:, ky:ky + 1] * jnp.cos((twopi * freqs[:, ky:ky + 1]) * ty)
    w = env * wave_x * wave_y
    col = colors_ref[...]
    part = jnp.concatenate(
        [jnp.sum(w * col[:, c0:c0 + 1], axis=0, keepdims=True) for c0 in range(3)],
        axis=0)

    @pl.when(pl.program_id(0) == 0)
    def _init():
        out_ref[...] = jnp.zeros_like(out_ref)

    out_ref[...] += part


def kernel(x, gaussian_colors, gaussian_positions, gaussian_scales,
           gaussian_rotations, wave_coefficients):
    G = wave_coefficients.shape[0]
    N = x.shape[0]
    wave2 = wave_coefficients.reshape(2 * G, _F)
    Rb = 800
    vals, freqs = pl.pallas_call(
        _topk_body,
        grid=(2 * G // Rb,),
        in_specs=[pl.BlockSpec((Rb, _F), lambda i: (i, 0))],
        out_specs=[pl.BlockSpec((Rb, _K), lambda i: (i, 0)),
                   pl.BlockSpec((Rb, _K), lambda i: (i, 0))],
        out_shape=[jax.ShapeDtypeStruct((2 * G, _K), jnp.float32),
                   jax.ShapeDtypeStruct((2 * G, _K), jnp.float32)],
    )(wave2)
    vals2 = vals.reshape(G, 2 * _K)
    freqs2 = freqs.reshape(G, 2 * _K)
    xt = x.T
    Gb = 400
    out_t = pl.pallas_call(
        _render_body,
        grid=(G // Gb,),
        in_specs=[
            pl.BlockSpec((2, N), lambda i: (0, 0)),
            pl.BlockSpec((Gb, 3), lambda i: (i, 0)),
            pl.BlockSpec((Gb, 2), lambda i: (i, 0)),
            pl.BlockSpec((Gb, 2), lambda i: (i, 0)),
            pl.BlockSpec((Gb, 1), lambda i: (i, 0)),
            pl.BlockSpec((Gb, 2 * _K), lambda i: (i, 0)),
            pl.BlockSpec((Gb, 2 * _K), lambda i: (i, 0)),
        ],
        out_specs=pl.BlockSpec((3, N), lambda i: (0, 0)),
        out_shape=jax.ShapeDtypeStruct((3, N), jnp.float32),
    )(xt, gaussian_colors, gaussian_positions, gaussian_scales,
      gaussian_rotations, vals2, freqs2)
    return out_t.T


# speedup vs baseline: 40.7334x; 2.9668x over previous
import jax
import jax.numpy as jnp
from jax.experimental import pallas as pl
from jax.experimental.pallas import tpu as pltpu

_K = 16
_F = 1024
_MAXF = 1024.0


_CP = (6.5286584, -25.9676, 60.167633, -85.45014, 64.93912, -19.739204, 1.0)


def _cos_2pi(u):
    v = u - jnp.round(u)
    z = v * v
    p = _CP[0] * z + _CP[1]
    for c in _CP[2:]:
        p = p * z + c
    return p


def _topk_body(w_ref, vals_ref, freqs_ref):
    v = w_ref[...]
    a = jnp.abs(v)
    iota = jax.lax.broadcasted_iota(jnp.int32, a.shape, 1)
    vals = []
    idxs = []
    for _ in range(_K):
        m = jnp.max(a, axis=1, keepdims=True)
        idx = jnp.min(jnp.where(a >= m, iota, _F), axis=1, keepdims=True)
        sel = iota == idx
        vals.append(jnp.sum(jnp.where(sel, v, 0.0), axis=1, keepdims=True))
        idxs.append(idx)
        a = jnp.where(sel, -1.0, a)
    vals_ref[...] = jnp.concatenate(vals, axis=1)
    freqs_ref[...] = (_MAXF / _F) * jnp.concatenate(idxs, axis=1).astype(jnp.float32)


def _render_body(xt_ref, colors_ref, pos_ref, scales_ref, rot_ref,
                 vals_ref, freqs_ref, out_ref):
    xx = xt_ref[0:1, :]
    xy = xt_ref[1:2, :]
    px = pos_ref[:, 0:1]
    py = pos_ref[:, 1:2]
    relx = xx - px
    rely = xy - py
    rot = rot_ref[:, 0:1]
    c = jnp.cos(rot)
    s = jnp.sin(rot)
    tx = c * relx + s * rely
    ty = -s * relx + c * rely
    sx = scales_ref[:, 0:1]
    sy = scales_ref[:, 1:2]
    env = jnp.exp(-0.5 * ((tx * sx) ** 2 + (ty * sy) ** 2))
    vals = vals_ref[...]
    freqs = freqs_ref[...]
    wave_x = jnp.zeros_like(tx)
    wave_y = jnp.zeros_like(ty)
    for k in range(_K):
        wave_x = wave_x + vals[:, k:k + 1] * _cos_2pi(freqs[:, k:k + 1] * tx)
        ky = _K + k
        wave_y = wave_y + vals[:, ky:ky + 1] * _cos_2pi(freqs[:, ky:ky + 1] * ty)
    w = env * wave_x * wave_y
    col = colors_ref[...]
    part = jnp.concatenate(
        [jnp.sum(w * col[:, c0:c0 + 1], axis=0, keepdims=True) for c0 in range(3)],
        axis=0)

    @pl.when(pl.program_id(0) == 0)
    def _init():
        out_ref[...] = jnp.zeros_like(out_ref)

    out_ref[...] += part


def kernel(x, gaussian_colors, gaussian_positions, gaussian_scales,
           gaussian_rotations, wave_coefficients):
    G = wave_coefficients.shape[0]
    N = x.shape[0]
    wave2 = wave_coefficients.reshape(2 * G, _F)
    Rb = 800
    vals, freqs = pl.pallas_call(
        _topk_body,
        grid=(2 * G // Rb,),
        in_specs=[pl.BlockSpec((Rb, _F), lambda i: (i, 0))],
        out_specs=[pl.BlockSpec((Rb, _K), lambda i: (i, 0)),
                   pl.BlockSpec((Rb, _K), lambda i: (i, 0))],
        out_shape=[jax.ShapeDtypeStruct((2 * G, _K), jnp.float32),
                   jax.ShapeDtypeStruct((2 * G, _K), jnp.float32)],
    )(wave2)
    vals2 = vals.reshape(G, 2 * _K)
    freqs2 = freqs.reshape(G, 2 * _K)
    xt = x.T
    Gb = 400
    out_t = pl.pallas_call(
        _render_body,
        grid=(G // Gb,),
        in_specs=[
            pl.BlockSpec((2, N), lambda i: (0, 0)),
            pl.BlockSpec((Gb, 3), lambda i: (i, 0)),
            pl.BlockSpec((Gb, 2), lambda i: (i, 0)),
            pl.BlockSpec((Gb, 2), lambda i: (i, 0)),
            pl.BlockSpec((Gb, 1), lambda i: (i, 0)),
            pl.BlockSpec((Gb, 2 * _K), lambda i: (i, 0)),
            pl.BlockSpec((Gb, 2 * _K), lambda i: (i, 0)),
        ],
        out_specs=pl.BlockSpec((3, N), lambda i: (0, 0)),
        out_shape=jax.ShapeDtypeStruct((3, N), jnp.float32),
    )(xt, gaussian_colors, gaussian_positions, gaussian_scales,
      gaussian_rotations, vals2, freqs2)
    return out_t.T
